# Initial kernel scaffold; baseline (speedup 1.0000x reference)
import jax
import jax.numpy as jnp
from jax.experimental import pallas as pl


def kernel(x, edge_index, W1, a_src1, a_dst1, b1, W2, a_src2, a_dst2, b2):
    raise NotImplementedError("write your pallas kernel here")


# speedup vs baseline: 20.6778x; 20.6778x over previous
import functools

import jax
import jax.numpy as jnp
from jax import lax
from jax.experimental import pallas as pl
from jax.experimental.pallas import tpu as pltpu
from jax.experimental.pallas import tpu_sc as plsc

N = 10000
E = 320000
NROWS = 640


def _dense1_body(x_ref, w_ref, a_ref, h_ref, asad_ref):
    h = jnp.dot(x_ref[...], w_ref[...], preferred_element_type=jnp.float32)
    h_ref[...] = h
    asad_ref[...] = jnp.dot(h, a_ref[...], preferred_element_type=jnp.float32)


def _dense1(x, w, a2, block_rows=1000):
    n, d_in = x.shape
    d_out = w.shape[1]
    return pl.pallas_call(
        _dense1_body,
        grid=(n // block_rows,),
        in_specs=[
            pl.BlockSpec((block_rows, d_in), lambda i: (i, 0)),
            pl.BlockSpec((d_in, d_out), lambda i: (0, 0)),
            pl.BlockSpec((d_out, 2), lambda i: (0, 0)),
        ],
        out_specs=[
            pl.BlockSpec((block_rows, d_out), lambda i: (i, 0)),
            pl.BlockSpec((block_rows, 2), lambda i: (i, 0)),
        ],
        out_shape=[
            jax.ShapeDtypeStruct((n, d_out), jnp.float32),
            jax.ShapeDtypeStruct((n, 2), jnp.float32),
        ],
    )(x, w, a2)


def _dense2_body(p0_ref, p1_ref, b_ref, w_ref, a_ref, h_ref, asad_ref):
    hin = jnp.maximum(p0_ref[...] + p1_ref[...] + b_ref[...], 0.0)
    h = jnp.dot(hin, w_ref[...], preferred_element_type=jnp.float32)
    h_ref[...] = h
    asad_ref[...] = jnp.dot(h, a_ref[...], preferred_element_type=jnp.float32)


def _dense2(p0, p1, b, w, a2, block_rows=1000):
    n, d_in = p0.shape
    d_out = w.shape[1]
    return pl.pallas_call(
        _dense2_body,
        grid=(n // block_rows,),
        in_specs=[
            pl.BlockSpec((block_rows, d_in), lambda i: (i, 0)),
            pl.BlockSpec((block_rows, d_in), lambda i: (i, 0)),
            pl.BlockSpec((1, d_in), lambda i: (0, 0)),
            pl.BlockSpec((d_in, d_out), lambda i: (0, 0)),
            pl.BlockSpec((d_out, 2), lambda i: (0, 0)),
        ],
        out_specs=[
            pl.BlockSpec((block_rows, d_out), lambda i: (i, 0)),
            pl.BlockSpec((block_rows, 2), lambda i: (i, 0)),
        ],
        out_shape=[
            jax.ShapeDtypeStruct((n, d_out), jnp.float32),
            jax.ShapeDtypeStruct((n, 2), jnp.float32),
        ],
    )(p0, p1, b, w, a2)


def _combine_body(p0_ref, p1_ref, b_ref, o_ref):
    d = o_ref.shape[1]
    o_ref[...] = p0_ref[:, :d] + p1_ref[:, :d] + b_ref[...]


def _combine(p0, p1, b, block_rows=1000):
    n, dp = p0.shape
    d = b.shape[1]
    return pl.pallas_call(
        _combine_body,
        grid=(n // block_rows,),
        in_specs=[
            pl.BlockSpec((block_rows, dp), lambda i: (i, 0)),
            pl.BlockSpec((block_rows, dp), lambda i: (i, 0)),
            pl.BlockSpec((1, d), lambda i: (0, 0)),
        ],
        out_specs=pl.BlockSpec((block_rows, d), lambda i: (i, 0)),
        out_shape=jax.ShapeDtypeStruct((n, d), jnp.float32),
    )(p0, p1, b)


def _make_sc_layer(d: int):
    mesh = plsc.VectorSubcoreMesh(core_axis_name="c", subcore_axis_name="s")
    EA = E // 16
    CA = 400
    NCA = EA // CA
    GA = CA // 16
    EB = E // 32
    CB = 80
    NCB = EB // CB
    GB = CB // 16
    RPT = N // 16

    @functools.partial(
        pl.kernel,
        out_type=jax.ShapeDtypeStruct((2, N, d), jnp.float32),
        mesh=mesh,
        compiler_params=pltpu.CompilerParams(needs_layout_passes=False),
        scratch_types=[
            pltpu.VMEM((N,), jnp.float32),
            pltpu.VMEM((N,), jnp.float32),
            pltpu.VMEM((80, 128), jnp.float32),
            pltpu.VMEM((25, d), jnp.float32),
            pltpu.VMEM((CA,), jnp.int32),
            pltpu.VMEM((CA,), jnp.int32),
            pltpu.VMEM((CB,), jnp.int32),
            pltpu.VMEM((CB,), jnp.int32),
            pltpu.VMEM((CB,), jnp.float32),
            pltpu.VMEM((CB, d), jnp.float32),
            pltpu.VMEM((1, 80), jnp.int32),
            pltpu.VMEM_SHARED((80, 128), jnp.float32),
            pltpu.VMEM_SHARED((N, d), jnp.float32),
            pltpu.SemaphoreType.DMA,
        ],
    )
    def sc_layer(h_hbm, src_hbm, dst_hbm, as_hbm, ad_hbm, out_hbm,
                 as_v, ad_v, s_v, zb, srcc, dstc, sidx, didx, alpha_v,
                 rows_v, id_ref, s_sh, out_sh, sem):
        c = lax.axis_index("c")
        s = lax.axis_index("s")
        wid = s * 2 + c

        def _zs(i, carry):
            for cc in range(8):
                s_v[i, pl.ds(cc * 16, 16)] = jnp.zeros((16,), jnp.float32)
            return carry
        lax.fori_loop(0, 80, _zs, 0)

        def _zz(i, carry):
            for cc in range(d // 16):
                zb[i, pl.ds(cc * 16, 16)] = jnp.zeros((16,), jnp.float32)
            return carry
        lax.fori_loop(0, 25, _zz, 0)

        pltpu.sync_copy(as_hbm, as_v)
        pltpu.sync_copy(ad_hbm, ad_v)

        for i in range(5):
            id_ref[0, pl.ds(i * 16, 16)] = lax.iota(jnp.int32, 16) + i * 16

        @pl.when(s == 0)
        def _():
            pltpu.sync_copy(s_v, s_sh)
        for j in range(RPT // 25):
            pltpu.sync_copy(zb, out_sh.at[pl.ds(s * RPT + j * 25, 25), :])

        def _chunk_a(i, carry):
            base = s * EA + i * CA
            pltpu.sync_copy(src_hbm.at[pl.ds(base, CA)], srcc)
            pltpu.sync_copy(dst_hbm.at[pl.ds(base, CA)], dstc)

            def _grp(g, carry2):
                si = srcc[pl.ds(g * 16, 16)]
                di = dstc[pl.ds(g * 16, 16)]
                e = plsc.load_gather(as_v, [si]) + plsc.load_gather(ad_v, [di])
                e = jnp.where(e >= 0.0, e, 0.2 * e)
                ex = jnp.exp(e)
                plsc.addupdate_scatter(s_v, [di >> 7, di & 127], ex)
                return carry2
            lax.fori_loop(0, GA, _grp, 0)
            return carry
        lax.fori_loop(0, NCA, _chunk_a, 0)

        plsc.subcore_barrier()
        pltpu.sync_copy(s_v, s_sh.at[id_ref.at[0]], add=True)
        plsc.subcore_barrier()
        pltpu.sync_copy(s_sh, s_v)

        def _chunk_b(i, carry):
            base = wid * EB + i * CB
            pltpu.sync_copy(src_hbm.at[pl.ds(base, CB)], sidx)
            pltpu.sync_copy(dst_hbm.at[pl.ds(base, CB)], didx)
            cp = pltpu.async_copy(h_hbm.at[sidx], rows_v, sem)
            for g in range(GB):
                si = sidx[pl.ds(g * 16, 16)]
                di = didx[pl.ds(g * 16, 16)]
                e = plsc.load_gather(as_v, [si]) + plsc.load_gather(ad_v, [di])
                e = jnp.where(e >= 0.0, e, 0.2 * e)
                ex = jnp.exp(e)
                sden = plsc.load_gather(s_v, [di >> 7, di & 127])
                alpha_v[pl.ds(g * 16, 16)] = ex / (sden + 1e-16)
            cp.wait()

            def _scale(g, carry2):
                a16 = alpha_v[pl.ds(g * 16, 16)]
                for j in range(16):
                    a = a16[j]
                    r = g * 16 + j
                    for cc in range(d // 16):
                        rows_v[r, pl.ds(cc * 16, 16)] = (
                            rows_v[r, pl.ds(cc * 16, 16)] * a)
                return carry2
            lax.fori_loop(0, GB, _scale, 0)
            pltpu.sync_copy(rows_v, out_sh.at[didx], add=True)
            return carry
        lax.fori_loop(0, NCB, _chunk_b, 0)

        plsc.subcore_barrier()
        r0 = pl.multiple_of(s * 624, 8)

        @pl.when(s < 15)
        def _():
            pltpu.sync_copy(out_sh.at[pl.ds(r0, 624), :],
                            out_hbm.at[c, pl.ds(r0, 624), :])

        @pl.when(s == 15)
        def _():
            pltpu.sync_copy(out_sh.at[pl.ds(9360, 640), :],
                            out_hbm.at[c, pl.ds(9360, 640), :])

    return sc_layer


_sc_layer_128 = _make_sc_layer(128)


def kernel(x, edge_index, W1, a_src1, a_dst1, b1, W2, a_src2, a_dst2, b2):
    ei = edge_index.astype(jnp.int32)
    src = ei[0]
    dst = ei[1]
    A1 = jnp.stack([a_src1, a_dst1], axis=1)
    W2p = jnp.pad(W2, ((0, 0), (0, 128 - W2.shape[1])))
    A2p = jnp.pad(jnp.stack([a_src2, a_dst2], axis=1),
                  ((0, 128 - W2.shape[1]), (0, 0)))

    h1, asad1 = _dense1(x, W1, A1)
    p1 = _sc_layer_128(h1, src, dst, asad1[:, 0], asad1[:, 1])
    h2, asad2 = _dense2(p1[0], p1[1], b1.reshape(1, -1), W2p, A2p)
    p2 = _sc_layer_128(h2, src, dst, asad2[:, 0], asad2[:, 1])
    return _combine(p2[0], p2[1], b2.reshape(1, -1))


# speedup vs baseline: 32.6457x; 1.5788x over previous
import functools

import jax
import jax.numpy as jnp
from jax import lax
from jax.experimental import pallas as pl
from jax.experimental.pallas import tpu as pltpu
from jax.experimental.pallas import tpu_sc as plsc

N = 10000
E = 320000
D = 128


def _dense1_body(x_ref, w_ref, a_ref, h_ref, asad_ref):
    h = jnp.dot(x_ref[...], w_ref[...], preferred_element_type=jnp.float32)
    h_ref[...] = h
    asad_ref[...] = jnp.dot(h, a_ref[...], preferred_element_type=jnp.float32)


def _dense1(x, w, a2, block_rows=1000):
    n, d_in = x.shape
    d_out = w.shape[1]
    return pl.pallas_call(
        _dense1_body,
        grid=(n // block_rows,),
        in_specs=[
            pl.BlockSpec((block_rows, d_in), lambda i: (i, 0)),
            pl.BlockSpec((d_in, d_out), lambda i: (0, 0)),
            pl.BlockSpec((d_out, 2), lambda i: (0, 0)),
        ],
        out_specs=[
            pl.BlockSpec((block_rows, d_out), lambda i: (i, 0)),
            pl.BlockSpec((block_rows, 2), lambda i: (i, 0)),
        ],
        out_shape=[
            jax.ShapeDtypeStruct((n, d_out), jnp.float32),
            jax.ShapeDtypeStruct((n, 2), jnp.float32),
        ],
    )(x, w, a2)


def _dense2_body(p0_ref, p1_ref, b_ref, w_ref, a_ref, h_ref, asad_ref):
    hin = jnp.maximum(p0_ref[...] + p1_ref[...] + b_ref[...], 0.0)
    h = jnp.dot(hin, w_ref[...], preferred_element_type=jnp.float32)
    h_ref[...] = h
    asad_ref[...] = jnp.dot(h, a_ref[...], preferred_element_type=jnp.float32)


def _dense2(p0, p1, b, w, a2, block_rows=1000):
    n, d_in = p0.shape
    d_out = w.shape[1]
    return pl.pallas_call(
        _dense2_body,
        grid=(n // block_rows,),
        in_specs=[
            pl.BlockSpec((block_rows, d_in), lambda i: (i, 0)),
            pl.BlockSpec((block_rows, d_in), lambda i: (i, 0)),
            pl.BlockSpec((1, d_in), lambda i: (0, 0)),
            pl.BlockSpec((d_in, d_out), lambda i: (0, 0)),
            pl.BlockSpec((d_out, 2), lambda i: (0, 0)),
        ],
        out_specs=[
            pl.BlockSpec((block_rows, d_out), lambda i: (i, 0)),
            pl.BlockSpec((block_rows, 2), lambda i: (i, 0)),
        ],
        out_shape=[
            jax.ShapeDtypeStruct((n, d_out), jnp.float32),
            jax.ShapeDtypeStruct((n, 2), jnp.float32),
        ],
    )(p0, p1, b, w, a2)


def _combine_body(p0_ref, p1_ref, b_ref, o_ref):
    d = o_ref.shape[1]
    o_ref[...] = p0_ref[:, :d] + p1_ref[:, :d] + b_ref[...]


def _combine(p0, p1, b, block_rows=1000):
    n, dp = p0.shape
    d = b.shape[1]
    return pl.pallas_call(
        _combine_body,
        grid=(n // block_rows,),
        in_specs=[
            pl.BlockSpec((block_rows, dp), lambda i: (i, 0)),
            pl.BlockSpec((block_rows, dp), lambda i: (i, 0)),
            pl.BlockSpec((1, d), lambda i: (0, 0)),
        ],
        out_specs=pl.BlockSpec((block_rows, d), lambda i: (i, 0)),
        out_shape=jax.ShapeDtypeStruct((n, d), jnp.float32),
    )(p0, p1, b)


_MESH = plsc.VectorSubcoreMesh(core_axis_name="c", subcore_axis_name="s")
EA = E // 16
EB = E // 32
SLAB = 2000
CB = 80
CPS = SLAB // CB
NSA = EA // SLAB
NSB = EB // SLAB


@functools.partial(
    pl.kernel,
    out_type=(
        jax.ShapeDtypeStruct((2, N, D), jnp.float32),
        jax.ShapeDtypeStruct((2, E // SLAB, CPS, CB), jnp.float32),
    ),
    mesh=_MESH,
    compiler_params=pltpu.CompilerParams(needs_layout_passes=False),
    scratch_types=[
        pltpu.VMEM((CB, D), jnp.float32),
        pltpu.VMEM((CB, D), jnp.float32),
        pltpu.VMEM((80, 128), jnp.float32),
        pltpu.VMEM((SLAB,), jnp.int32),
        pltpu.VMEM((CPS, CB), jnp.int32),
        pltpu.VMEM((CPS, CB), jnp.float32),
        pltpu.VMEM((1, 80), jnp.int32),
        pltpu.VMEM_SHARED((80, 128), jnp.float32),
        pltpu.VMEM_SHARED((N, D), jnp.float32),
        pltpu.SemaphoreType.DMA,
        pltpu.SemaphoreType.DMA,
        pltpu.SemaphoreType.DMA,
        pltpu.SemaphoreType.DMA,
    ],
)
def _sc_layer(h_hbm, src_hbm, dst3_hbm, as2_hbm, ad2_hbm, out_hbm, ex_hbm,
              rows0, rows1, s_v, sidx_slab, didx_slab, exc_slab, id_ref,
              s_sh, out_sh, semg0, semg1, sems0, sems1):
    c = lax.axis_index("c")
    s = lax.axis_index("s")
    wid = s * 2 + c

    def _z80(ref):
        def body(i, carry):
            for cc in range(8):
                ref[i, pl.ds(cc * 16, 16)] = jnp.zeros((16,), jnp.float32)
            return carry
        lax.fori_loop(0, 80, body, 0)

    _z80(s_v)
    _z80(rows0)

    @pl.when(s == 0)
    def _():
        pltpu.sync_copy(s_v, s_sh)
    for j in range(7):
        pltpu.sync_copy(rows0, out_sh.at[pl.ds(s * 625 + j * 80, 80), :])
    pltpu.sync_copy(rows0.at[pl.ds(0, 65), :],
                    out_sh.at[pl.ds(s * 625 + 560, 65), :])

    for i in range(5):
        id_ref[0, pl.ds(i * 16, 16)] = lax.iota(jnp.int32, 16) + i * 16

    pltpu.sync_copy(as2_hbm, rows0)
    pltpu.sync_copy(ad2_hbm, rows1)

    for j in range(NSA):
        base = s * EA + j * SLAB
        cp1 = pltpu.async_copy(src_hbm.at[pl.ds(base, SLAB)], sidx_slab,
                               semg0)
        cp2 = pltpu.async_copy(dst3_hbm.at[2 * s + (1 if j >= 5 else 0),
                                           j % 5], didx_slab, semg1)
        cp1.wait()
        cp2.wait()

        def _arow(r, carry):
            for gg in range(5):
                si = sidx_slab[pl.ds(r * CB + gg * 16, 16)]
                di = didx_slab[r, pl.ds(gg * 16, 16)]
                e = (plsc.load_gather(rows0, [si >> 7, si & 127])
                     + plsc.load_gather(rows1, [di >> 7, di & 127]))
                e = jnp.where(e >= 0.0, e, 0.2 * e)
                ex = jnp.exp(e)
                exc_slab[r, pl.ds(gg * 16, 16)] = ex
                plsc.addupdate_scatter(s_v, [di >> 7, di & 127], ex)
            return carry
        lax.fori_loop(0, CPS, _arow, 0)
        pltpu.sync_copy(exc_slab, ex_hbm.at[c, s * NSA + j])

    plsc.subcore_barrier()
    pltpu.sync_copy(s_v, s_sh.at[id_ref.at[0]], add=True)
    plsc.subcore_barrier()
    pltpu.sync_copy(s_sh, s_v)

    rows = (rows0, rows1)
    semg = (semg0, semg1)
    sems = (sems0, sems1)

    def _gather(k, b):
        return pltpu.async_copy(
            h_hbm.at[sidx_slab.at[pl.ds(k * CB, CB)]], rows[b], semg[b])

    def _wait_gather(b):
        pltpu.make_async_copy(
            h_hbm.at[sidx_slab.at[pl.ds(0, CB)]], rows[b], semg[b]).wait()

    def _scatter(k, b):
        return pltpu.async_copy(rows[b], out_sh.at[didx_slab.at[k]],
                                sems[b], add=True)

    def _wait_scatter(b):
        pltpu.make_async_copy(rows[b], out_sh.at[didx_slab.at[0]],
                              sems[b]).wait()

    def _scale(k, b):
        def gbody(gg, carry):
            di = didx_slab[k, pl.ds(gg * 16, 16)]
            exv = exc_slab[k, pl.ds(gg * 16, 16)]
            sden = plsc.load_gather(s_v, [di >> 7, di & 127])
            al = exv / (sden + 1e-16)
            for jj in range(16):
                a = al[jj]
                r = gg * 16 + jj
                for cc in range(8):
                    rows[b][r, pl.ds(cc * 16, 16)] = (
                        rows[b][r, pl.ds(cc * 16, 16)] * a)
            return carry
        lax.fori_loop(0, 5, gbody, 0)

    for m in range(NSB):
        base = wid * EB + m * SLAB
        cp1 = pltpu.async_copy(src_hbm.at[pl.ds(base, SLAB)], sidx_slab,
                               semg0)
        cp2 = pltpu.async_copy(dst3_hbm.at[wid, m], didx_slab, semg1)
        cp3 = pltpu.async_copy(ex_hbm.at[c, wid * NSB + m], exc_slab,
                               sems0)
        cp1.wait()
        cp2.wait()
        cp3.wait()
        A = m % 2
        B = 1 - A
        _gather(0, A)

        def _pair(t, carry):
            _wait_gather(A)
            _scale(2 * t, A)

            @pl.when(t >= 1)
            def _():
                _wait_scatter(B)
            _gather(2 * t + 1, B)
            _scatter(2 * t, A)
            _wait_gather(B)
            _scale(2 * t + 1, B)
            _wait_scatter(A)
            _gather(2 * t + 2, A)
            _scatter(2 * t + 1, B)
            return carry
        lax.fori_loop(0, 12, _pair, 0)
        _wait_gather(A)
        _scale(24, A)
        _wait_scatter(B)
        _scatter(24, A)
        _wait_scatter(A)

    plsc.subcore_barrier()
    r0 = pl.multiple_of(s * 624, 8)

    @pl.when(s < 15)
    def _():
        pltpu.sync_copy(out_sh.at[pl.ds(r0, 624), :],
                        out_hbm.at[c, pl.ds(r0, 624), :])

    @pl.when(s == 15)
    def _():
        pltpu.sync_copy(out_sh.at[pl.ds(9360, 640), :],
                        out_hbm.at[c, pl.ds(9360, 640), :])


def kernel(x, edge_index, W1, a_src1, a_dst1, b1, W2, a_src2, a_dst2, b2):
    ei = edge_index.astype(jnp.int32)
    src = ei[0]
    dst = ei[1]
    dst3 = dst.reshape(32, 5, CPS, CB)
    A1 = jnp.stack([a_src1, a_dst1], axis=1)
    W2p = jnp.pad(W2, ((0, 0), (0, 128 - W2.shape[1])))
    A2p = jnp.pad(jnp.stack([a_src2, a_dst2], axis=1),
                  ((0, 128 - W2.shape[1]), (0, 0)))

    def _a2d(v):
        return jnp.pad(v, (0, 80 * 128 - N)).reshape(80, 128)

    h1, asad1 = _dense1(x, W1, A1)
    p1, _ = _sc_layer(h1, src, dst3, _a2d(asad1[:, 0]), _a2d(asad1[:, 1]))
    h2, asad2 = _dense2(p1[0], p1[1], b1.reshape(1, -1), W2p, A2p)
    p2, _ = _sc_layer(h2, src, dst3, _a2d(asad2[:, 0]), _a2d(asad2[:, 1]))
    return _combine(p2[0], p2[1], b2.reshape(1, -1))


# speedup vs baseline: 32.9349x; 1.0089x over previous
import functools

import jax
import jax.numpy as jnp
from jax import lax
from jax.experimental import pallas as pl
from jax.experimental.pallas import tpu as pltpu
from jax.experimental.pallas import tpu_sc as plsc

N = 10000
E = 320000
D = 128


def _dense1_body(x_ref, w_ref, a_ref, h_ref, asad_ref):
    h = jnp.dot(x_ref[...], w_ref[...], preferred_element_type=jnp.float32)
    h_ref[...] = h
    asad_ref[...] = jnp.dot(h, a_ref[...], preferred_element_type=jnp.float32)


def _dense1(x, w, a2, block_rows=1000):
    n, d_in = x.shape
    d_out = w.shape[1]
    return pl.pallas_call(
        _dense1_body,
        grid=(n // block_rows,),
        in_specs=[
            pl.BlockSpec((block_rows, d_in), lambda i: (i, 0)),
            pl.BlockSpec((d_in, d_out), lambda i: (0, 0)),
            pl.BlockSpec((d_out, 2), lambda i: (0, 0)),
        ],
        out_specs=[
            pl.BlockSpec((block_rows, d_out), lambda i: (i, 0)),
            pl.BlockSpec((block_rows, 2), lambda i: (i, 0)),
        ],
        out_shape=[
            jax.ShapeDtypeStruct((n, d_out), jnp.float32),
            jax.ShapeDtypeStruct((n, 2), jnp.float32),
        ],
    )(x, w, a2)


def _dense2_body(p0_ref, p1_ref, b_ref, w_ref, a_ref, h_ref, asad_ref):
    hin = jnp.maximum(p0_ref[...] + p1_ref[...] + b_ref[...], 0.0)
    h = jnp.dot(hin, w_ref[...], preferred_element_type=jnp.float32)
    h_ref[...] = h
    asad_ref[...] = jnp.dot(h, a_ref[...], preferred_element_type=jnp.float32)


def _dense2(p0, p1, b, w, a2, block_rows=1000):
    n, d_in = p0.shape
    d_out = w.shape[1]
    return pl.pallas_call(
        _dense2_body,
        grid=(n // block_rows,),
        in_specs=[
            pl.BlockSpec((block_rows, d_in), lambda i: (i, 0)),
            pl.BlockSpec((block_rows, d_in), lambda i: (i, 0)),
            pl.BlockSpec((1, d_in), lambda i: (0, 0)),
            pl.BlockSpec((d_in, d_out), lambda i: (0, 0)),
            pl.BlockSpec((d_out, 2), lambda i: (0, 0)),
        ],
        out_specs=[
            pl.BlockSpec((block_rows, d_out), lambda i: (i, 0)),
            pl.BlockSpec((block_rows, 2), lambda i: (i, 0)),
        ],
        out_shape=[
            jax.ShapeDtypeStruct((n, d_out), jnp.float32),
            jax.ShapeDtypeStruct((n, 2), jnp.float32),
        ],
    )(p0, p1, b, w, a2)


def _combine_body(p0_ref, p1_ref, b_ref, o_ref):
    d = o_ref.shape[1]
    o_ref[...] = p0_ref[:, :d] + p1_ref[:, :d] + b_ref[...]


def _combine(p0, p1, b, block_rows=1000):
    n, dp = p0.shape
    d = b.shape[1]
    return pl.pallas_call(
        _combine_body,
        grid=(n // block_rows,),
        in_specs=[
            pl.BlockSpec((block_rows, dp), lambda i: (i, 0)),
            pl.BlockSpec((block_rows, dp), lambda i: (i, 0)),
            pl.BlockSpec((1, d), lambda i: (0, 0)),
        ],
        out_specs=pl.BlockSpec((block_rows, d), lambda i: (i, 0)),
        out_shape=jax.ShapeDtypeStruct((n, d), jnp.float32),
    )(p0, p1, b)


_MESH = plsc.VectorSubcoreMesh(core_axis_name="c", subcore_axis_name="s")
EA = E // 16
EB = E // 32
SLAB = 2000
CB = 80
CPS = SLAB // CB
NSA = EA // SLAB
NSB = EB // SLAB


@functools.partial(
    pl.kernel,
    out_type=jax.ShapeDtypeStruct((2, N, D), jnp.float32),
    mesh=_MESH,
    compiler_params=pltpu.CompilerParams(needs_layout_passes=False),
    scratch_types=[
        pltpu.VMEM((CB, D), jnp.float32),
        pltpu.VMEM((CB, D), jnp.float32),
        pltpu.VMEM((80, 128), jnp.float32),
        pltpu.VMEM((SLAB,), jnp.int32),
        pltpu.VMEM((CPS, CB), jnp.int32),
        pltpu.VMEM((EB,), jnp.float32),
        pltpu.VMEM((1, 80), jnp.int32),
        pltpu.VMEM_SHARED((80, 128), jnp.float32),
        pltpu.VMEM_SHARED((N, D), jnp.float32),
        pltpu.SemaphoreType.DMA,
        pltpu.SemaphoreType.DMA,
        pltpu.SemaphoreType.DMA,
        pltpu.SemaphoreType.DMA,
    ],
)
def _sc_layer(h_hbm, src_hbm, dst3_hbm, as2_hbm, ad2_hbm, out_hbm,
              rows0, rows1, s_v, sidx_slab, didx_slab, exB, id_ref,
              s_sh, out_sh, semg0, semg1, sems0, sems1):
    c = lax.axis_index("c")
    s = lax.axis_index("s")
    wid = s * 2 + c

    def _z80(ref):
        def body(i, carry):
            for cc in range(8):
                ref[i, pl.ds(cc * 16, 16)] = jnp.zeros((16,), jnp.float32)
            return carry
        lax.fori_loop(0, 80, body, 0)

    _z80(s_v)
    _z80(rows0)

    @pl.when(s == 0)
    def _():
        pltpu.sync_copy(s_v, s_sh)
    for j in range(7):
        pltpu.sync_copy(rows0, out_sh.at[pl.ds(s * 625 + j * 80, 80), :])
    pltpu.sync_copy(rows0.at[pl.ds(0, 65), :],
                    out_sh.at[pl.ds(s * 625 + 560, 65), :])

    for i in range(5):
        id_ref[0, pl.ds(i * 16, 16)] = lax.iota(jnp.int32, 16) + i * 16

    pltpu.sync_copy(as2_hbm, rows0)
    pltpu.sync_copy(ad2_hbm, rows1)

    for j in range(NSA):
        JHALF = 0 if j < 5 else 1
        JOFF = j % 5
        base = s * EA + j * SLAB
        cp1 = pltpu.async_copy(src_hbm.at[pl.ds(base, SLAB)], sidx_slab,
                               semg0)
        cp2 = pltpu.async_copy(dst3_hbm.at[2 * s + (1 if j >= 5 else 0),
                                           j % 5], didx_slab, semg1)
        cp1.wait()
        cp2.wait()

        def _arow(r, carry):
            for gg in range(5):
                si = sidx_slab[pl.ds(r * CB + gg * 16, 16)]
                di = didx_slab[r, pl.ds(gg * 16, 16)]
                e = (plsc.load_gather(rows0, [si >> 7, si & 127])
                     + plsc.load_gather(rows1, [di >> 7, di & 127]))
                e = jnp.where(e >= 0.0, e, 0.2 * e)
                ex = jnp.exp(e)

                @pl.when(c == JHALF)
                def _():
                    exB[pl.ds(JOFF * SLAB + r * CB + gg * 16, 16)] = ex
                plsc.addupdate_scatter(s_v, [di >> 7, di & 127], ex)
            return carry
        lax.fori_loop(0, CPS, _arow, 0)

    plsc.subcore_barrier()
    pltpu.sync_copy(s_v, s_sh.at[id_ref.at[0]], add=True)
    plsc.subcore_barrier()
    pltpu.sync_copy(s_sh, s_v)

    rows = (rows0, rows1)
    semg = (semg0, semg1)
    sems = (sems0, sems1)

    def _gather(k, b):
        return pltpu.async_copy(
            h_hbm.at[sidx_slab.at[pl.ds(k * CB, CB)]], rows[b], semg[b])

    def _wait_gather(b):
        pltpu.make_async_copy(
            h_hbm.at[sidx_slab.at[pl.ds(0, CB)]], rows[b], semg[b]).wait()

    def _scatter(k, b):
        return pltpu.async_copy(rows[b], out_sh.at[didx_slab.at[k]],
                                sems[b], add=True)

    def _wait_scatter(b):
        pltpu.make_async_copy(rows[b], out_sh.at[didx_slab.at[0]],
                              sems[b]).wait()

    def _scale(k, b, MOFF):
        def gbody(gg, carry):
            di = didx_slab[k, pl.ds(gg * 16, 16)]
            exv = exB[pl.ds(MOFF * SLAB + k * CB + gg * 16, 16)]
            sden = plsc.load_gather(s_v, [di >> 7, di & 127])
            al = exv / (sden + 1e-16)
            for jj in range(16):
                a = al[jj]
                r = gg * 16 + jj
                for cc in range(8):
                    rows[b][r, pl.ds(cc * 16, 16)] = (
                        rows[b][r, pl.ds(cc * 16, 16)] * a)
            return carry
        lax.fori_loop(0, 5, gbody, 0)

    for m in range(NSB):
        MOFF = m
        base = wid * EB + m * SLAB
        cp1 = pltpu.async_copy(src_hbm.at[pl.ds(base, SLAB)], sidx_slab,
                               semg0)
        cp2 = pltpu.async_copy(dst3_hbm.at[wid, m], didx_slab, semg1)
        cp1.wait()
        cp2.wait()
        A = m % 2
        B = 1 - A
        _gather(0, A)

        def _pair(t, carry):
            _wait_gather(A)
            _scale(2 * t, A, MOFF)

            @pl.when(t >= 1)
            def _():
                _wait_scatter(B)
            _gather(2 * t + 1, B)
            _scatter(2 * t, A)
            _wait_gather(B)
            _scale(2 * t + 1, B, MOFF)
            _wait_scatter(A)
            _gather(2 * t + 2, A)
            _scatter(2 * t + 1, B)
            return carry
        lax.fori_loop(0, 12, _pair, 0)
        _wait_gather(A)
        _scale(24, A, MOFF)
        _wait_scatter(B)
        _scatter(24, A)
        _wait_scatter(A)

    plsc.subcore_barrier()
    r0 = pl.multiple_of(s * 624, 8)

    @pl.when(s < 15)
    def _():
        pltpu.sync_copy(out_sh.at[pl.ds(r0, 624), :],
                        out_hbm.at[c, pl.ds(r0, 624), :])

    @pl.when(s == 15)
    def _():
        pltpu.sync_copy(out_sh.at[pl.ds(9360, 640), :],
                        out_hbm.at[c, pl.ds(9360, 640), :])


def kernel(x, edge_index, W1, a_src1, a_dst1, b1, W2, a_src2, a_dst2, b2):
    ei = edge_index.astype(jnp.int32)
    src = ei[0]
    dst = ei[1]
    dst3 = dst.reshape(32, 5, CPS, CB)
    A1 = jnp.stack([a_src1, a_dst1], axis=1)
    W2p = jnp.pad(W2, ((0, 0), (0, 128 - W2.shape[1])))
    A2p = jnp.pad(jnp.stack([a_src2, a_dst2], axis=1),
                  ((0, 128 - W2.shape[1]), (0, 0)))

    def _a2d(v):
        return jnp.pad(v, (0, 80 * 128 - N)).reshape(80, 128)

    h1, asad1 = _dense1(x, W1, A1)
    p1 = _sc_layer(h1, src, dst3, _a2d(asad1[:, 0]), _a2d(asad1[:, 1]))
    h2, asad2 = _dense2(p1[0], p1[1], b1.reshape(1, -1), W2p, A2p)
    p2 = _sc_layer(h2, src, dst3, _a2d(asad2[:, 0]), _a2d(asad2[:, 1]))
    return _combine(p2[0], p2[1], b2.reshape(1, -1))
